# baseline (speedup 1.0000x reference)
import functools

import jax
import jax.numpy as jnp
from jax import lax
from jax.experimental import pallas as pl
from jax.experimental.pallas import tpu as pltpu
from jax.experimental.pallas import tpu_sc as plsc

NUM_TABLE_ROWS = 30
EMBED_DIM = 64
PAIR_DIM = 2 * EMBED_DIM
NUM_PT_ROWS = NUM_TABLE_ROWS * NUM_TABLE_ROWS
NUM_INDICES = 4096 * 200
NUM_PAIRS = NUM_INDICES // 2
NUM_CORES = 2
NUM_SUBCORES = 16
NUM_WORKERS = NUM_CORES * NUM_SUBCORES
P_PER_W = NUM_PAIRS // NUM_WORKERS
CHUNK_P = 160
CHUNK_R = 2 * CHUNK_P
NCHUNK = P_PER_W // CHUNK_P
LANES = 16
NBUF = 2
UNROLL = 8

_mesh = plsc.VectorSubcoreMesh(core_axis_name="c", subcore_axis_name="s")

_DNUMS = lax.GatherDimensionNumbers(
    offset_dims=(), collapsed_slice_dims=(0,), start_index_map=(0,))


def _dg(vec, idx):
    return lax.gather(vec, idx[:, None], _DNUMS, (1,),
                      mode=lax.GatherScatterMode.PROMISE_IN_BOUNDS)


@functools.partial(
    pl.kernel,
    out_type=jax.ShapeDtypeStruct((NUM_INDICES, EMBED_DIM), jnp.float32),
    mesh=_mesh,
    scratch_types=[
        pltpu.VMEM_SHARED((NUM_PT_ROWS, PAIR_DIM), jnp.float32),
        [pltpu.VMEM((2 * CHUNK_P,), jnp.int32) for _ in range(NBUF)],
        pltpu.VMEM((CHUNK_P,), jnp.int32),
        pltpu.VMEM((CHUNK_P, PAIR_DIM), jnp.float32),
        [pltpu.VMEM((CHUNK_R, EMBED_DIM), jnp.float32) for _ in range(NBUF)],
        [pltpu.SemaphoreType.DMA for _ in range(NBUF)],
        pltpu.SemaphoreType.DMA,
        [pltpu.SemaphoreType.DMA for _ in range(NBUF)],
    ],
)
def _embed_gather(idx_hbm, pt_hbm, out_hbm, pt_sh, idx_v, pair_v, buf_c,
                  rows_v, i_s, g_s, s_s):
    sid = lax.axis_index("s")
    wid = sid * NUM_CORES + lax.axis_index("c")
    base_p = wid * P_PER_W

    @pl.when(sid == 0)
    def _():
        pltpu.sync_copy(pt_hbm, pt_sh)
    plsc.subcore_barrier()

    lane = lax.iota(jnp.int32, LANES)
    perm_ev = (lane * 2) & (LANES - 1)
    perm_od = perm_ev + 1
    lo_half = lane < (LANES // 2)

    def pair_off(g):
        return pl.multiple_of(base_p + g * CHUNK_P, 16)

    def start_idx(g, b):
        off = pair_off(g)
        pltpu.async_copy(idx_hbm.at[pl.ds(off * 2, 2 * CHUNK_P)], idx_v[b],
                         i_s[b])

    for b in range(NBUF):
        start_idx(b, b)

    def process(g, b):
        off = pair_off(g)
        pltpu.make_async_copy(idx_hbm.at[pl.ds(off * 2, 2 * CHUNK_P)],
                              idx_v[b], i_s[b]).wait()
        for j in range(CHUNK_P // LANES):
            v0 = idx_v[b][pl.ds(2 * j * LANES, LANES)]
            v1 = idx_v[b][pl.ds((2 * j + 1) * LANES, LANES)]
            ev = jnp.where(lo_half, _dg(v0, perm_ev), _dg(v1, perm_ev))
            od = jnp.where(lo_half, _dg(v0, perm_od), _dg(v1, perm_od))
            pair_v[pl.ds(j * LANES, LANES)] = ev * NUM_TABLE_ROWS + od
        @pl.when(g + NBUF < NCHUNK)
        def _():
            start_idx(g + NBUF, b)
        pltpu.async_copy(pt_sh.at[pair_v], buf_c, g_s).wait()
        @pl.when(g >= NBUF)
        def _():
            pltpu.make_async_copy(rows_v[b],
                                  out_hbm.at[pl.ds(off * 2, CHUNK_R)],
                                  s_s[b]).wait()
        def expand(i, carry):
            for k in range(UNROLL):
                p = i * UNROLL + k
                for j in range(4):
                    s = pl.ds(j * LANES, LANES)
                    rows_v[b][2 * p, s] = buf_c[p, s]
                for j in range(4):
                    s = pl.ds(j * LANES, LANES)
                    rows_v[b][2 * p + 1, s] = buf_c[p, pl.ds((j + 4) * LANES,
                                                             LANES)]
            return carry
        lax.fori_loop(0, CHUNK_P // UNROLL, expand, 0)
        pltpu.async_copy(rows_v[b], out_hbm.at[pl.ds(off * 2, CHUNK_R)],
                         s_s[b])

    def body(i, carry):
        for b in range(NBUF):
            process(i * NBUF + b, b)
        return carry

    lax.fori_loop(0, NCHUNK // NBUF, body, 0)

    for b in range(NBUF):
        off = pair_off(NCHUNK - NBUF + b)
        pltpu.make_async_copy(rows_v[b], out_hbm.at[pl.ds(off * 2, CHUNK_R)],
                              s_s[b]).wait()


def kernel(types, table):
    flat = types.reshape(-1)
    left = jnp.repeat(table, NUM_TABLE_ROWS, axis=0)
    right = jnp.tile(table, (NUM_TABLE_ROWS, 1))
    paired = jnp.concatenate([left, right], axis=1)
    out = _embed_gather(flat, paired)
    return out.reshape(types.shape + (EMBED_DIM,))


# speedup vs baseline: 1.4066x; 1.4066x over previous
import functools

import jax
import jax.numpy as jnp
from jax import lax
from jax.experimental import pallas as pl
from jax.experimental.pallas import tpu as pltpu
from jax.experimental.pallas import tpu_sc as plsc

NUM_TABLE_ROWS = 30
EMBED_DIM = 64
NUM_INDICES = 4096 * 200
NUM_CORES = 2
NUM_SUBCORES = 16
NUM_WORKERS = NUM_CORES * NUM_SUBCORES
R_PER_W = NUM_INDICES // NUM_WORKERS
CHUNK = 400
NCHUNK = R_PER_W // CHUNK
NBUF = 2

_mesh = plsc.VectorSubcoreMesh(core_axis_name="c", subcore_axis_name="s")


@functools.partial(
    pl.kernel,
    out_type=jax.ShapeDtypeStruct((4096, 200, EMBED_DIM), jnp.float32),
    mesh=_mesh,
    scratch_types=[
        pltpu.VMEM_SHARED((NUM_TABLE_ROWS, EMBED_DIM), jnp.float32),
        pltpu.VMEM((NUM_TABLE_ROWS * EMBED_DIM,), jnp.float32),
        [pltpu.VMEM((CHUNK,), jnp.int32) for _ in range(NBUF)],
        [pltpu.VMEM((CHUNK, EMBED_DIM), jnp.float32) for _ in range(NBUF)],
        [pltpu.SemaphoreType.DMA for _ in range(NBUF)],
        pltpu.SemaphoreType.DMA,
        [pltpu.SemaphoreType.DMA for _ in range(NBUF)],
    ],
)
def _embed_gather(idx_hbm, tb_hbm, out_hbm, tb_sh, tb1d_v, idx_v, rows_v,
                  i_s, g_s, s_s):
    sid = lax.axis_index("s")
    wid = sid * NUM_CORES + lax.axis_index("c")
    base = wid * R_PER_W
    out2 = out_hbm.reshape(NUM_INDICES, EMBED_DIM)

    @pl.when(sid == 0)
    def _():
        pltpu.sync_copy(tb_hbm, tb1d_v)
        for r in range(NUM_TABLE_ROWS):
            pltpu.sync_copy(tb1d_v.at[pl.ds(r * EMBED_DIM, EMBED_DIM)],
                            tb_sh.at[r])
    plsc.subcore_barrier()

    _run_chunks(idx_hbm, out2, tb_sh, idx_v, rows_v, i_s, g_s, s_s, base)


def _run_chunks(idx_hbm, out2, tb, idx_v, rows_v, i_s, g_s, s_s, base):
    def chunk_off(g):
        return pl.multiple_of(base + g * CHUNK, 16)

    def start_idx(g, b):
        off = chunk_off(g)
        pltpu.async_copy(idx_hbm.at[pl.ds(off, CHUNK)], idx_v[b], i_s[b])

    for b in range(NBUF):
        start_idx(b, b)

    def process(g, b):
        off = chunk_off(g)
        pltpu.make_async_copy(idx_hbm.at[pl.ds(off, CHUNK)],
                              idx_v[b], i_s[b]).wait()
        @pl.when(g + NBUF < NCHUNK)
        def _():
            start_idx(g + NBUF, b)
        @pl.when(g >= NBUF)
        def _():
            pltpu.make_async_copy(rows_v[b], out2.at[pl.ds(off, CHUNK)],
                                  s_s[b]).wait()
        pltpu.async_copy(tb.at[idx_v[b]], rows_v[b], g_s).wait()
        pltpu.async_copy(rows_v[b], out2.at[pl.ds(off, CHUNK)], s_s[b])

    def loop_body(i, carry):
        for b in range(NBUF):
            process(i * NBUF + b, b)
        return carry

    lax.fori_loop(0, NCHUNK // NBUF, loop_body, 0)

    for b in range(NBUF):
        off = chunk_off(NCHUNK - NBUF + b)
        pltpu.make_async_copy(rows_v[b], out2.at[pl.ds(off, CHUNK)],
                              s_s[b]).wait()


def kernel(types, table):
    flat = types.reshape(-1)
    return _embed_gather(flat, table.reshape(-1))


# speedup vs baseline: 1.8300x; 1.3010x over previous
import functools

import jax
import jax.numpy as jnp
from jax import lax
from jax.experimental import pallas as pl
from jax.experimental.pallas import tpu as pltpu
from jax.experimental.pallas import tpu_sc as plsc

NUM_TABLE_ROWS = 30
EMBED_DIM = 64
NUM_INDICES = 4096 * 200
NUM_CORES = 2
NUM_SUBCORES = 16
NUM_WORKERS = NUM_CORES * NUM_SUBCORES
R_PER_W = NUM_INDICES // NUM_WORKERS
CHUNK = 400
NCHUNK = R_PER_W // CHUNK
NBUF = 2

_mesh = plsc.VectorSubcoreMesh(core_axis_name="c", subcore_axis_name="s")


@functools.partial(
    pl.kernel,
    out_type=jax.ShapeDtypeStruct((NUM_INDICES, EMBED_DIM), jnp.float32),
    mesh=_mesh,
    scratch_types=[
        pltpu.VMEM_SHARED((NUM_TABLE_ROWS, EMBED_DIM), jnp.float32),
        pltpu.VMEM((NUM_TABLE_ROWS * EMBED_DIM,), jnp.float32),
        [pltpu.VMEM((CHUNK,), jnp.int32) for _ in range(NBUF)],
        [pltpu.VMEM((CHUNK, EMBED_DIM), jnp.float32) for _ in range(NBUF)],
        [pltpu.SemaphoreType.DMA for _ in range(NBUF)],
        pltpu.SemaphoreType.DMA,
        [pltpu.SemaphoreType.DMA for _ in range(NBUF)],
    ],
)
def _embed_gather(idx_hbm, tb_hbm, out_hbm, tb_sh, tb1d_v, idx_v, rows_v,
                  i_s, g_s, s_s):
    sid = lax.axis_index("s")
    wid = sid * NUM_CORES + lax.axis_index("c")
    base = wid * R_PER_W
    out2 = out_hbm

    @pl.when(sid == 0)
    def _():
        pltpu.sync_copy(tb_hbm, tb1d_v)
        for r in range(NUM_TABLE_ROWS):
            pltpu.sync_copy(tb1d_v.at[pl.ds(r * EMBED_DIM, EMBED_DIM)],
                            tb_sh.at[r])
    plsc.subcore_barrier()

    _run_chunks(idx_hbm, out2, tb_sh, idx_v, rows_v, i_s, g_s, s_s, base)


def _run_chunks(idx_hbm, out2, tb, idx_v, rows_v, i_s, g_s, s_s, base):
    def chunk_off(g):
        return pl.multiple_of(base + g * CHUNK, 16)

    def start_idx(g, b):
        off = chunk_off(g)
        pltpu.async_copy(idx_hbm.at[pl.ds(off, CHUNK)], idx_v[b], i_s[b])

    for b in range(NBUF):
        start_idx(b, b)

    def process(g, b):
        off = chunk_off(g)
        pltpu.make_async_copy(idx_hbm.at[pl.ds(off, CHUNK)],
                              idx_v[b], i_s[b]).wait()
        @pl.when(g + NBUF < NCHUNK)
        def _():
            start_idx(g + NBUF, b)
        @pl.when(g >= NBUF)
        def _():
            pltpu.make_async_copy(rows_v[b], out2.at[pl.ds(off, CHUNK)],
                                  s_s[b]).wait()
        pltpu.async_copy(tb.at[idx_v[b]], rows_v[b], g_s).wait()
        pltpu.async_copy(rows_v[b], out2.at[pl.ds(off, CHUNK)], s_s[b])

    def loop_body(i, carry):
        for b in range(NBUF):
            process(i * NBUF + b, b)
        return carry

    lax.fori_loop(0, NCHUNK // NBUF, loop_body, 0)

    for b in range(NBUF):
        off = chunk_off(NCHUNK - NBUF + b)
        pltpu.make_async_copy(rows_v[b], out2.at[pl.ds(off, CHUNK)],
                              s_s[b]).wait()


def kernel(types, table):
    flat = types.reshape(-1)
    out = _embed_gather(flat, table.reshape(-1))
    return out.reshape(types.shape + (EMBED_DIM,))
